# Initial kernel scaffold; baseline (speedup 1.0000x reference)
import functools

import jax
import jax.numpy as jnp
from jax import lax
from jax.experimental import pallas as pl
from jax.experimental.pallas import tpu as pltpu
from jax.experimental.pallas import tpu_sc as plsc

TABLE = 1_000_000
N_IND = 16384 * 100
NC, NS, L = 2, 16, 16
NW = NC * NS
PER_W = N_IND // NW
CLAMP_ITERS = PER_W // L


def _body(ind_hbm, w_hbm, out_hbm, w_sp, idx_v, val_v, sem):
    cid = lax.axis_index("c")
    sid = lax.axis_index("s")
    wid = sid * NC + cid

    @pl.when(sid == 0)
    def _stage():
        pltpu.sync_copy(w_hbm, w_sp)

    plsc.subcore_barrier()

    base = wid * PER_W
    pltpu.sync_copy(ind_hbm.at[pl.ds(base, PER_W)], idx_v)
    pltpu.async_copy(w_sp.at[idx_v], val_v, sem).wait()

    lo = jnp.float32(1e-12)
    hi = jnp.float32(1.0)

    def clamp(i, _):
        sl = pl.ds(i * L, L)
        val_v[sl] = jnp.minimum(jnp.maximum(val_v[sl], lo), hi)
        return 0

    lax.fori_loop(0, CLAMP_ITERS, clamp, 0)

    pltpu.sync_copy(val_v, out_hbm.at[pl.ds(base, PER_W)])


def kernel(ind, w):
    ind_flat = ind.reshape(N_IND).astype(jnp.int32)
    w_flat = w.reshape(TABLE)
    mesh = plsc.VectorSubcoreMesh(core_axis_name="c", subcore_axis_name="s")
    out = pl.kernel(
        _body,
        out_type=jax.ShapeDtypeStruct((N_IND,), jnp.float32),
        mesh=mesh,
        scratch_types=[
            pltpu.VMEM_SHARED((TABLE,), jnp.float32),
            pltpu.VMEM((PER_W,), jnp.int32),
            pltpu.VMEM((PER_W,), jnp.float32),
            pltpu.SemaphoreType.DMA,
        ],
    )(ind_flat, w_flat)
    return out.reshape(16384, 100)


# speedup vs baseline: 1.0821x; 1.0821x over previous
import functools

import jax
import jax.numpy as jnp
from jax import lax
from jax.experimental import pallas as pl
from jax.experimental.pallas import tpu as pltpu
from jax.experimental.pallas import tpu_sc as plsc

TABLE = 1_000_000
N_IND = 16384 * 100
NC, NS, L = 2, 16, 16
NW = NC * NS
PER_W = N_IND // NW
CHUNK = 12_800
NCHUNK = PER_W // CHUNK
CLAMP_ITERS = CHUNK // L


def _body(ind_hbm, w_hbm, out_hbm, w_sp, idx_v, val_v, sem):
    cid = lax.axis_index("c")
    sid = lax.axis_index("s")
    wid = sid * NC + cid

    @pl.when(sid == 0)
    def _stage():
        pltpu.sync_copy(w_hbm, w_sp)

    plsc.subcore_barrier()

    base = wid * PER_W
    lo = jnp.float32(1e-12)
    hi = jnp.float32(1.0)

    for k in range(NCHUNK):
        off = base + k * CHUNK
        pltpu.sync_copy(ind_hbm.at[pl.ds(off, CHUNK)], idx_v)
        pltpu.async_copy(w_sp.at[idx_v], val_v, sem).wait()

        def clamp(i, _):
            sl = pl.ds(i * L, L)
            val_v[sl] = jnp.minimum(jnp.maximum(val_v[sl], lo), hi)
            return 0

        lax.fori_loop(0, CLAMP_ITERS, clamp, 0)

        pltpu.sync_copy(val_v, out_hbm.at[pl.ds(off, CHUNK)])


def kernel(ind, w):
    ind_flat = ind.reshape(N_IND).astype(jnp.int32)
    w_flat = w.reshape(TABLE)
    mesh = plsc.VectorSubcoreMesh(core_axis_name="c", subcore_axis_name="s")
    out = pl.kernel(
        _body,
        out_type=jax.ShapeDtypeStruct((N_IND,), jnp.float32),
        mesh=mesh,
        scratch_types=[
            pltpu.VMEM_SHARED((TABLE,), jnp.float32),
            pltpu.VMEM((CHUNK,), jnp.int32),
            pltpu.VMEM((CHUNK,), jnp.float32),
            pltpu.SemaphoreType.DMA,
        ],
    )(ind_flat, w_flat)
    return out.reshape(16384, 100)


# speedup vs baseline: 1.3248x; 1.2242x over previous
import jax
import jax.numpy as jnp
from jax import lax
from jax.experimental import pallas as pl
from jax.experimental.pallas import tpu as pltpu
from jax.experimental.pallas import tpu_sc as plsc

TABLE = 1_000_000
ROWS, COLS = 16384, 100
NC, NS, L = 2, 16, 16
NW = NC * NS
ROWS_W = ROWS // NW
RCHUNK = 128
NCHUNK = ROWS_W // RCHUNK
FAN = 8
COL_OFFS = (0, 16, 32, 48, 64, 80, 84)


def _body(ind_hbm, w_hbm, out_hbm, w_sp, idx_v, val_v, sem):
    cid = lax.axis_index("c")
    sid = lax.axis_index("s")
    wid = sid * NC + cid

    @pl.when(sid == 0)
    def _stage():
        pltpu.sync_copy(w_hbm, w_sp)

    plsc.subcore_barrier()

    r0 = wid * ROWS_W
    lo = jnp.float32(1e-12)
    hi = jnp.float32(1.0)

    for k in range(NCHUNK):
        rows = pl.ds(r0 + k * RCHUNK, RCHUNK)
        pltpu.sync_copy(ind_hbm.at[rows], idx_v)

        def gather(i, _):
            b = i * FAN
            copies = [
                pltpu.async_copy(w_sp.at[idx_v.at[b + j]], val_v.at[b + j], sem)
                for j in range(FAN)
            ]
            for c in copies:
                c.wait()
            return 0

        lax.fori_loop(0, RCHUNK // FAN, gather, 0)

        def clamp(r, _):
            for c in COL_OFFS:
                sl = pl.ds(c, L)
                val_v[r, sl] = jnp.minimum(jnp.maximum(val_v[r, sl], lo), hi)
            return 0

        lax.fori_loop(0, RCHUNK, clamp, 0)

        pltpu.sync_copy(val_v, out_hbm.at[rows])


def kernel(ind, w):
    ind = ind.astype(jnp.int32)
    w_flat = w.reshape(TABLE)
    mesh = plsc.VectorSubcoreMesh(core_axis_name="c", subcore_axis_name="s")
    out = pl.kernel(
        _body,
        out_type=jax.ShapeDtypeStruct((ROWS, COLS), jnp.float32),
        mesh=mesh,
        scratch_types=[
            pltpu.VMEM_SHARED((TABLE,), jnp.float32),
            pltpu.VMEM((RCHUNK, COLS), jnp.int32),
            pltpu.VMEM((RCHUNK, COLS), jnp.float32),
            pltpu.SemaphoreType.DMA,
        ],
    )(ind, w_flat)
    return out
